# baseline (device time: 46373 ns/iter reference)
import os

import jax
import jax.numpy as jnp
from jax import lax
from jax.experimental import pallas as pl
from jax.experimental.pallas import tpu as pltpu

N_DEV = 4
N_HOP = N_DEV - 1
SUB = int(os.environ.get("KSUB", "2"))
_MODE = os.environ.get("KMODE", "full")
_WIRE = os.environ.get("KWIRE", "bf16")


def kernel(x, w_mat):
    m_per, k = x.shape
    _, n_per = w_mat.shape
    m_half = m_per // 2
    m_sub = m_half // SUB
    wire_dtype = jnp.bfloat16 if _WIRE == "bf16" else jnp.float32

    def body(x_ref, w_ref, out_ref,
             xs_ref, cw_ref, ccw_ref, cw_send, cw_recv, ccw_send, ccw_recv):
        my_pos = lax.axis_index("i")
        left = lax.rem(my_pos - 1 + N_DEV, N_DEV)
        right = lax.rem(my_pos + 1, N_DEV)

        if wire_dtype != jnp.float32:
            xs_ref[...] = x_ref[...].astype(wire_dtype)

        barrier_sem = pltpu.get_barrier_semaphore()
        for nbr in (left, right):
            pl.semaphore_signal(
                barrier_sem, inc=1,
                device_id=(nbr,), device_id_type=pl.DeviceIdType.MESH,
            )
        pl.semaphore_wait(barrier_sem, 2)

        hop0_src = x_ref if wire_dtype == jnp.float32 else xs_ref

        def sub_rdma(h, s):
            sub_slice = pl.ds(s * m_sub, m_sub)
            if h == 0:
                cw_src = hop0_src.at[pl.ds(s * m_sub, m_sub)]
                ccw_src = hop0_src.at[pl.ds(m_half + s * m_sub, m_sub)]
            else:
                cw_src = cw_ref.at[h - 1, sub_slice]
                ccw_src = ccw_ref.at[h - 1, sub_slice]
            cw = pltpu.make_async_remote_copy(
                src_ref=cw_src, dst_ref=cw_ref.at[h, sub_slice],
                send_sem=cw_send.at[h, s], recv_sem=cw_recv.at[h, s],
                device_id=(right,), device_id_type=pl.DeviceIdType.MESH,
            )
            ccw = pltpu.make_async_remote_copy(
                src_ref=ccw_src, dst_ref=ccw_ref.at[h, sub_slice],
                send_sem=ccw_send.at[h, s], recv_sem=ccw_recv.at[h, s],
                device_id=(left,), device_id_type=pl.DeviceIdType.MESH,
            )
            return cw, ccw

        def gemm_store(buf, out_row, rows):
            out_ref[pl.ds(out_row, rows), :] = jnp.maximum(
                jnp.dot(buf.astype(jnp.float32), w_ref[...],
                        preferred_element_type=jnp.float32),
                0.0,
            )

        def hop_compute(h, sub=None):
            cw_origin = lax.rem(my_pos - h - 1 + 2 * N_DEV, N_DEV)
            ccw_origin = lax.rem(my_pos + h + 1, N_DEV)
            if sub is None:
                gemm_store(cw_ref[h], cw_origin * m_per, m_half)
                gemm_store(ccw_ref[h], ccw_origin * m_per + m_half, m_half)
            else:
                s, direction = sub
                if direction == "cw":
                    gemm_store(cw_ref[h, pl.ds(s * m_sub, m_sub)],
                               cw_origin * m_per + s * m_sub, m_sub)
                else:
                    gemm_store(ccw_ref[h, pl.ds(s * m_sub, m_sub)],
                               ccw_origin * m_per + m_half + s * m_sub, m_sub)

        do_comm = _MODE != "compute"
        do_compute = _MODE != "comm"
        started = []

        if do_comm:
            hop0 = [sub_rdma(0, s) for s in range(SUB)]
            for cw, ccw in hop0:
                cw.start()
                ccw.start()
                started.append((cw, ccw))
            prev = hop0

        if do_compute:
            out_ref[pl.ds(my_pos * m_per, m_per), :] = jnp.maximum(
                jnp.dot(x_ref[...], w_ref[...],
                        preferred_element_type=jnp.float32),
                0.0,
            )

        for h in range(1, N_HOP):
            if do_comm:
                cur = []
                for s in range(SUB):
                    pcw, pccw = prev[s]
                    cw, ccw = sub_rdma(h, s)
                    pcw.wait_recv()
                    cw.start()
                    pccw.wait_recv()
                    ccw.start()
                    cur.append((cw, ccw))
                    started.append((cw, ccw))
                prev = cur
            if do_compute:
                hop_compute(h - 1)

        for s in range(SUB):
            if do_comm:
                cw, ccw = prev[s]
                cw.wait_recv()
            if do_compute:
                hop_compute(N_HOP - 1, sub=(s, "cw"))
            if do_comm:
                ccw.wait_recv()
            if do_compute:
                hop_compute(N_HOP - 1, sub=(s, "ccw"))

        for cw, ccw in started:
            cw.wait_send()
            ccw.wait_send()

    return pl.pallas_call(
        body,
        out_shape=jax.ShapeDtypeStruct((N_DEV * m_per, n_per), jnp.float32),
        in_specs=[
            pl.BlockSpec(memory_space=pltpu.VMEM),
            pl.BlockSpec(memory_space=pltpu.VMEM),
        ],
        out_specs=pl.BlockSpec(memory_space=pltpu.VMEM),
        scratch_shapes=[
            pltpu.VMEM((m_per, k), wire_dtype),
            pltpu.VMEM((N_HOP, m_half, k), wire_dtype),
            pltpu.VMEM((N_HOP, m_half, k), wire_dtype),
            pltpu.SemaphoreType.DMA((N_HOP, SUB)),
            pltpu.SemaphoreType.DMA((N_HOP, SUB)),
            pltpu.SemaphoreType.DMA((N_HOP, SUB)),
            pltpu.SemaphoreType.DMA((N_HOP, SUB)),
        ],
        compiler_params=pltpu.CompilerParams(collective_id=0),
    )(x, w_mat)


# device time: 45324 ns/iter; 1.0231x vs baseline; 1.0231x over previous
import os

import jax
import jax.numpy as jnp
from jax import lax
from jax.experimental import pallas as pl
from jax.experimental.pallas import tpu as pltpu

N_DEV = 4
N_HOP = N_DEV - 1
SUB = int(os.environ.get("KSUB", "2"))
_MODE = os.environ.get("KMODE", "full")
_WIRE = os.environ.get("KWIRE", "bf16")


def kernel(x, w_mat):
    m_per, k = x.shape
    _, n_per = w_mat.shape
    m_half = m_per // 2
    m_sub = m_half // SUB
    wire_dtype = jnp.bfloat16 if _WIRE == "bf16" else jnp.float32

    def body(x_ref, w_ref, out_ref,
             xs_ref, cw_ref, ccw_ref, cw_send, cw_recv, ccw_send, ccw_recv):
        my_pos = lax.axis_index("i")
        left = lax.rem(my_pos - 1 + N_DEV, N_DEV)
        right = lax.rem(my_pos + 1, N_DEV)

        barrier_sem = pltpu.get_barrier_semaphore()
        for nbr in (left, right):
            pl.semaphore_signal(
                barrier_sem, inc=1,
                device_id=(nbr,), device_id_type=pl.DeviceIdType.MESH,
            )
        pl.semaphore_wait(barrier_sem, 2)

        hop0_src = x_ref if wire_dtype == jnp.float32 else xs_ref

        def sub_rdma(h, s):
            sub_slice = pl.ds(s * m_sub, m_sub)
            if h == 0:
                cw_src = hop0_src.at[pl.ds(s * m_sub, m_sub)]
                ccw_src = hop0_src.at[pl.ds(m_half + s * m_sub, m_sub)]
            else:
                cw_src = cw_ref.at[h - 1, sub_slice]
                ccw_src = ccw_ref.at[h - 1, sub_slice]
            cw = pltpu.make_async_remote_copy(
                src_ref=cw_src, dst_ref=cw_ref.at[h, sub_slice],
                send_sem=cw_send.at[h, s], recv_sem=cw_recv.at[h, s],
                device_id=(right,), device_id_type=pl.DeviceIdType.MESH,
            )
            ccw = pltpu.make_async_remote_copy(
                src_ref=ccw_src, dst_ref=ccw_ref.at[h, sub_slice],
                send_sem=ccw_send.at[h, s], recv_sem=ccw_recv.at[h, s],
                device_id=(left,), device_id_type=pl.DeviceIdType.MESH,
            )
            return cw, ccw

        def gemm_store(buf, out_row, rows):
            out_ref[pl.ds(out_row, rows), :] = jnp.maximum(
                jnp.dot(buf.astype(jnp.float32), w_ref[...],
                        preferred_element_type=jnp.float32),
                0.0,
            )

        def hop_compute(h, sub=None):
            cw_origin = lax.rem(my_pos - h - 1 + 2 * N_DEV, N_DEV)
            ccw_origin = lax.rem(my_pos + h + 1, N_DEV)
            if sub is None:
                gemm_store(cw_ref[h], cw_origin * m_per, m_half)
                gemm_store(ccw_ref[h], ccw_origin * m_per + m_half, m_half)
            else:
                s, direction = sub
                if direction == "cw":
                    gemm_store(cw_ref[h, pl.ds(s * m_sub, m_sub)],
                               cw_origin * m_per + s * m_sub, m_sub)
                else:
                    gemm_store(ccw_ref[h, pl.ds(s * m_sub, m_sub)],
                               ccw_origin * m_per + m_half + s * m_sub, m_sub)

        do_comm = _MODE != "compute"
        do_compute = _MODE != "comm"
        started = []

        if do_comm:
            hop0 = []
            for s in range(SUB):
                if wire_dtype != jnp.float32:
                    cw_sl = pl.ds(s * m_sub, m_sub)
                    ccw_sl = pl.ds(m_half + s * m_sub, m_sub)
                    xs_ref[cw_sl, :] = x_ref[cw_sl, :].astype(wire_dtype)
                    xs_ref[ccw_sl, :] = x_ref[ccw_sl, :].astype(wire_dtype)
                cw, ccw = sub_rdma(0, s)
                cw.start()
                ccw.start()
                hop0.append((cw, ccw))
                started.append((cw, ccw))
            prev = hop0

        if do_compute:
            out_ref[pl.ds(my_pos * m_per, m_per), :] = jnp.maximum(
                jnp.dot(x_ref[...], w_ref[...],
                        preferred_element_type=jnp.float32),
                0.0,
            )

        for h in range(1, N_HOP):
            if do_comm:
                cur = []
                for s in range(SUB):
                    pcw, pccw = prev[s]
                    cw, ccw = sub_rdma(h, s)
                    pcw.wait_recv()
                    cw.start()
                    pccw.wait_recv()
                    ccw.start()
                    cur.append((cw, ccw))
                    started.append((cw, ccw))
                prev = cur
            if do_compute:
                hop_compute(h - 1)

        for s in range(SUB):
            if do_comm:
                cw, ccw = prev[s]
                cw.wait_recv()
            if do_compute:
                hop_compute(N_HOP - 1, sub=(s, "cw"))
            if do_comm:
                ccw.wait_recv()
            if do_compute:
                hop_compute(N_HOP - 1, sub=(s, "ccw"))

        for cw, ccw in started:
            cw.wait_send()
            ccw.wait_send()

    return pl.pallas_call(
        body,
        out_shape=jax.ShapeDtypeStruct((N_DEV * m_per, n_per), jnp.float32),
        in_specs=[
            pl.BlockSpec(memory_space=pltpu.VMEM),
            pl.BlockSpec(memory_space=pltpu.VMEM),
        ],
        out_specs=pl.BlockSpec(memory_space=pltpu.VMEM),
        scratch_shapes=[
            pltpu.VMEM((m_per, k), wire_dtype),
            pltpu.VMEM((N_HOP, m_half, k), wire_dtype),
            pltpu.VMEM((N_HOP, m_half, k), wire_dtype),
            pltpu.SemaphoreType.DMA((N_HOP, SUB)),
            pltpu.SemaphoreType.DMA((N_HOP, SUB)),
            pltpu.SemaphoreType.DMA((N_HOP, SUB)),
            pltpu.SemaphoreType.DMA((N_HOP, SUB)),
        ],
        compiler_params=pltpu.CompilerParams(collective_id=0),
    )(x, w_mat)
